# baseline (speedup 1.0000x reference)
import functools
import math

import jax
import jax.numpy as jnp
from jax import lax
from jax.experimental import pallas as pl
from jax.experimental.pallas import tpu as pltpu
from jax.experimental.pallas import tpu_sc as plsc

VOCAB = 1000000
EMB = 64
SCALE = math.sqrt(EMB)

NC = 2
NS = 16
NW = NC * NS
CH = 128


def _make_sc_kernel(n_chunks):
    mesh = plsc.VectorSubcoreMesh(core_axis_name="c", subcore_axis_name="s")

    @functools.partial(
        pl.kernel,
        mesh=mesh,
        out_type=jax.ShapeDtypeStruct((NW, n_chunks, CH, EMB), jnp.float32),
        scratch_types=[
            pltpu.VMEM((n_chunks, CH), jnp.int32),
            pltpu.VMEM((CH, EMB), jnp.float32),
            pltpu.SemaphoreType.DMA,
        ],
        compiler_params=pltpu.CompilerParams(use_tc_tiling_on_sc=False),
    )
    def sc_embed(tok_hbm, table_hbm, out_hbm, idx_v, rows_v, sem):
        wid = lax.axis_index("s") * NC + lax.axis_index("c")
        pltpu.sync_copy(tok_hbm.at[wid], idx_v)

        def chunk_body(g, carry):
            pltpu.async_copy(table_hbm.at[idx_v.at[g]], rows_v, sem).wait()

            def scale_body(r, c):
                for k in range(EMB // 16):
                    sl = pl.ds(k * 16, 16)
                    rows_v[r, sl] = rows_v[r, sl] * SCALE
                return c

            lax.fori_loop(0, CH, scale_body, 0, unroll=2)
            pltpu.sync_copy(rows_v, out_hbm.at[wid, g])
            return carry

        lax.fori_loop(0, n_chunks, chunk_body, 0)

    return sc_embed


def kernel(tokens, table):
    b, s = tokens.shape
    total = b * s
    n_chunks = total // (NW * CH)
    tok = tokens.reshape(NW, n_chunks, CH).astype(jnp.int32)
    out = _make_sc_kernel(n_chunks)(tok, table)
    return out.reshape(b, s, EMB)


# speedup vs baseline: 1.1472x; 1.1472x over previous
import functools
import math

import jax
import jax.numpy as jnp
from jax import lax
from jax.experimental import pallas as pl
from jax.experimental.pallas import tpu as pltpu
from jax.experimental.pallas import tpu_sc as plsc

EMB = 64
SCALE = math.sqrt(EMB)

NC = 2
NS = 16
NW = NC * NS
CH = 128
NBUF = 4


def _make_sc_kernel(n_chunks):
    assert n_chunks % NBUF == 0 and n_chunks >= 2 * NBUF
    n_rings = n_chunks // NBUF
    mesh = plsc.VectorSubcoreMesh(core_axis_name="c", subcore_axis_name="s")

    @functools.partial(
        pl.kernel,
        mesh=mesh,
        out_type=jax.ShapeDtypeStruct((NW, n_chunks, CH, EMB), jnp.float32),
        scratch_types=[
            pltpu.VMEM((n_chunks, CH), jnp.int32),
            pltpu.VMEM((NBUF, CH, EMB), jnp.float32),
        ]
        + [pltpu.SemaphoreType.DMA] * (2 * NBUF),
        compiler_params=pltpu.CompilerParams(use_tc_tiling_on_sc=False),
    )
    def sc_embed(tok_hbm, table_hbm, out_hbm, idx_v, rows_v, *sems):
        in_sem = sems[:NBUF]
        out_sem = sems[NBUF:]
        wid = lax.axis_index("s") * NC + lax.axis_index("c")
        pltpu.sync_copy(tok_hbm.at[wid], idx_v)

        def gather(g, b):
            return pltpu.make_async_copy(
                table_hbm.at[idx_v.at[g]], rows_v.at[b], in_sem[b])

        def scatter(g, b):
            return pltpu.make_async_copy(
                rows_v.at[b], out_hbm.at[wid, g], out_sem[b])

        gather(0, 0).start()
        gather(1, 1).start()

        def ring_body(i, carry):
            base = i * NBUF
            for b in range(NBUF):
                g = base + b
                gather(g, b).wait()
                rows = rows_v.at[b]

                @plsc.parallel_loop(0, CH, unroll=8)
                def _(r):
                    for k in range(EMB // 16):
                        sl = pl.ds(k * 16, 16)
                        rows[r, sl] = rows[r, sl] * SCALE

                scatter(g, b).start()
                bn = (b + 2) % NBUF
                if b < NBUF - 2:
                    gn = g + 2

                    @pl.when(i > 0)
                    def _():
                        scatter(gn - NBUF, bn).wait()

                    gather(gn, bn).start()
                else:
                    gn = g + 2

                    @pl.when(i < n_rings - 1)
                    def _():
                        scatter(gn - NBUF, bn).wait()
                        gather(gn, bn).start()
            return carry

        lax.fori_loop(0, n_rings, ring_body, 0)
        for g in range(n_chunks - 4, n_chunks):
            scatter(g, g % NBUF).wait()

    return sc_embed


def kernel(tokens, table):
    b, s = tokens.shape
    total = b * s
    n_chunks = total // (NW * CH)
    tok = tokens.reshape(NW, n_chunks, CH).astype(jnp.int32)
    out = _make_sc_kernel(n_chunks)(tok, table)
    return out.reshape(b, s, EMB)
